# Initial kernel scaffold; baseline (speedup 1.0000x reference)
import dataclasses
import functools

import jax
import jax.numpy as jnp
from jax import lax
from jax.experimental import pallas as pl
from jax.experimental.pallas import tpu as pltpu
from jax.experimental.pallas import tpu_sc as plsc

H = W = 224
HP = WP = 232
L1 = 8
N = 8
NBLK = (H // 8) * (W // 8)
ULEN = NBLK * 64
NUNITS = N * L1


def _codes_body(xp_ref, w1_ref, w2_ref, out_ref, o1_ref):
    l1 = pl.program_id(1)
    acc = jnp.zeros((H, W), jnp.float32)
    for c in range(3):
        for di in range(7):
            src = xp_ref[0, c, di:di + H, :]
            for dj in range(7):
                acc = acc + w1_ref[l1, c, di, dj] * src[:, dj:dj + W]
    o1_ref[...] = jnp.zeros((HP, WP), jnp.float32)
    o1_ref[3:3 + H, 3:3 + W] = acc
    code = jnp.zeros((H, W), jnp.int32)
    for l2 in range(2, 8):
        acc2 = jnp.zeros((H, W), jnp.float32)
        for di in range(7):
            src = o1_ref[di:di + H, :]
            for dj in range(7):
                acc2 = acc2 + w2_ref[l2, 0, di, dj] * src[:, dj:dj + W]
        code = code + jnp.where(acc2 > 0, jnp.int32(1 << (l2 - 2)), jnp.int32(0))
    r = lax.broadcasted_iota(jnp.int32, (H, W), 0)
    cc = lax.broadcasted_iota(jnp.int32, (H, W), 1)
    blk = (r >> 3) * (W // 8) + (cc >> 3)
    out_ref[0, 0] = code + (blk << 6)


def _codes_call(x_pad, w1, w2):
    return pl.pallas_call(
        _codes_body,
        grid=(N, L1),
        in_specs=[
            pl.BlockSpec((1, 3, HP, WP), lambda n, l: (n, 0, 0, 0)),
            pl.BlockSpec(memory_space=pltpu.SMEM),
            pl.BlockSpec(memory_space=pltpu.SMEM),
        ],
        out_specs=pl.BlockSpec((1, 1, H, W), lambda n, l: (n, l, 0, 0)),
        out_shape=jax.ShapeDtypeStruct((N, L1, H, W), jnp.int32),
        scratch_shapes=[pltpu.VMEM((HP, WP), jnp.float32)],
        compiler_params=pltpu.CompilerParams(
            dimension_semantics=("parallel", "arbitrary")),
    )(x_pad, w1, w2)


def _sc_hist_body(fidx_hbm, out_hbm, idx_ref, hist_ref):
    c = lax.axis_index("c")
    s = lax.axis_index("s")
    w = s * 2 + c
    zeros16 = jnp.zeros((16,), jnp.float32)
    ones16 = jnp.ones((16,), jnp.float32)
    for k in range(2):
        u = w + 32 * k
        pltpu.sync_copy(fidx_hbm.at[u], idx_ref)

        @pl.loop(0, ULEN, step=16)
        def _zero(i):
            hist_ref[pl.ds(i, 16)] = zeros16

        @pl.loop(0, ULEN, step=16)
        def _scat(i):
            v = idx_ref[pl.ds(i, 16)]
            plsc.addupdate_scatter(hist_ref, [v], ones16)

        pltpu.sync_copy(hist_ref, out_hbm.at[u])


def _hist_call(fidx):
    cp = pltpu.CompilerParams()
    if "needs_layout_passes" in pltpu.CompilerParams.__dataclass_fields__:
        cp = dataclasses.replace(cp, needs_layout_passes=False)
    mesh = plsc.VectorSubcoreMesh(core_axis_name="c", subcore_axis_name="s")
    f = pl.kernel(
        _sc_hist_body,
        out_type=jax.ShapeDtypeStruct((NUNITS, ULEN), jnp.float32),
        mesh=mesh,
        scratch_types=[
            pltpu.VMEM((ULEN,), jnp.int32),
            pltpu.VMEM((ULEN,), jnp.float32),
        ],
        compiler_params=cp,
    )
    return f(fidx)


@jax.jit
def kernel(x, w1, w2):
    x_pad = jnp.pad(x, ((0, 0), (0, 0), (3, 5), (3, 5)))
    codes = _codes_call(x_pad, w1, w2)
    hist = _hist_call(codes.reshape(NUNITS, ULEN))
    return hist.reshape(N, L1 * ULEN)


# speedup vs baseline: 6.4612x; 6.4612x over previous
import dataclasses
import functools

import jax
import jax.numpy as jnp
from jax import lax
from jax.experimental import pallas as pl
from jax.experimental.pallas import tpu as pltpu
from jax.experimental.pallas import tpu_sc as plsc

H = W = 224
HP = WP = 232
L1 = 8
N = 8
NBLK = (H // 8) * (W // 8)
ULEN = NBLK * 64
NUNITS = N * L1


def _codes_body(xp_ref, w1_ref, w2_ref, out_ref, o1_ref):
    l1 = pl.program_id(1)
    acc = jnp.zeros((H, W), jnp.float32)
    for c in range(3):
        for di in range(7):
            src = xp_ref[0, c, di:di + H, :]
            for dj in range(7):
                acc = acc + w1_ref[l1, c, di, dj] * src[:, dj:dj + W]
    acc = acc.astype(jnp.bfloat16).astype(jnp.float32)
    o1_ref[...] = jnp.zeros((HP, WP), jnp.float32)
    o1_ref[3:3 + H, 3:3 + W] = acc
    code = jnp.zeros((H, W), jnp.int32)
    for l2 in range(2, 8):
        acc2 = jnp.zeros((H, W), jnp.float32)
        for di in range(7):
            src = o1_ref[di:di + H, :]
            for dj in range(7):
                acc2 = acc2 + w2_ref[l2, 0, di, dj] * src[:, dj:dj + W]
        code = code + jnp.where(acc2 > 0, jnp.int32(1 << (l2 - 2)), jnp.int32(0))
    r = lax.broadcasted_iota(jnp.int32, (H, W), 0)
    cc = lax.broadcasted_iota(jnp.int32, (H, W), 1)
    blk = (r >> 3) * (W // 8) + (cc >> 3)
    out_ref[0, 0] = code + (blk << 6)


def _codes_call(x_pad, w1, w2):
    return pl.pallas_call(
        _codes_body,
        grid=(N, L1),
        in_specs=[
            pl.BlockSpec((1, 3, HP, WP), lambda n, l: (n, 0, 0, 0)),
            pl.BlockSpec(memory_space=pltpu.SMEM),
            pl.BlockSpec(memory_space=pltpu.SMEM),
        ],
        out_specs=pl.BlockSpec((1, 1, H, W), lambda n, l: (n, l, 0, 0)),
        out_shape=jax.ShapeDtypeStruct((N, L1, H, W), jnp.int32),
        scratch_shapes=[pltpu.VMEM((HP, WP), jnp.float32)],
        compiler_params=pltpu.CompilerParams(
            dimension_semantics=("parallel", "arbitrary")),
    )(x_pad, w1, w2)


def _sc_hist_body(fidx_hbm, out_hbm, idx_ref, hist_ref):
    c = lax.axis_index("c")
    s = lax.axis_index("s")
    w = s * 2 + c
    zeros16 = jnp.zeros((16,), jnp.float32)
    ones16 = jnp.ones((16,), jnp.float32)
    stride16 = lax.iota(jnp.int32, 16) * 64
    for k in range(2):
        u = w + 32 * k
        pltpu.sync_copy(fidx_hbm.at[u], idx_ref)

        @pl.loop(0, ULEN, step=16)
        def _zero(i):
            hist_ref[pl.ds(i, 16)] = zeros16

        @pl.loop(0, NBLK, step=16)
        def _grp(b):
            base = stride16 + b * 64

            @pl.loop(0, 64)
            def _px(j):
                v = plsc.load_gather(idx_ref, [base + j])
                plsc.addupdate_scatter(hist_ref, [v], ones16)

        pltpu.sync_copy(hist_ref, out_hbm.at[u])


def _hist_call(fidx):
    cp = pltpu.CompilerParams()
    if "needs_layout_passes" in pltpu.CompilerParams.__dataclass_fields__:
        cp = dataclasses.replace(cp, needs_layout_passes=False)
    mesh = plsc.VectorSubcoreMesh(core_axis_name="c", subcore_axis_name="s")
    f = pl.kernel(
        _sc_hist_body,
        out_type=jax.ShapeDtypeStruct((NUNITS, ULEN), jnp.float32),
        mesh=mesh,
        scratch_types=[
            pltpu.VMEM((ULEN,), jnp.int32),
            pltpu.VMEM((ULEN,), jnp.float32),
        ],
        compiler_params=cp,
    )
    return f(fidx)


@jax.jit
def kernel(x, w1, w2):
    xq = x.astype(jnp.bfloat16).astype(jnp.float32)
    w1q = w1.astype(jnp.bfloat16).astype(jnp.float32)
    w2q = w2.astype(jnp.bfloat16).astype(jnp.float32)
    x_pad = jnp.pad(xq, ((0, 0), (0, 0), (3, 5), (3, 5)))
    codes = _codes_call(x_pad, w1q, w2q)
    hist = _hist_call(codes.reshape(NUNITS, ULEN))
    return hist.reshape(N, L1 * ULEN)
